# baseline (speedup 1.0000x reference)
import functools

import jax
import jax.numpy as jnp
from jax import lax
from jax.experimental import pallas as pl
from jax.experimental.pallas import tpu as pltpu
from jax.experimental.pallas import tpu_sc as plsc

NC, NS, LANES = 2, 16, 16
NW = NC * NS
CHUNK = 32


def kernel(state, timestep, embed_table):
    B, L, D = state.shape
    N = B * L
    state_f = state.reshape(N, D)
    ts_f = timestep.reshape(N)
    rows_per_w = N // NW
    n_chunks = rows_per_w // CHUNK

    mesh = plsc.VectorSubcoreMesh(core_axis_name="c", subcore_axis_name="s")

    @functools.partial(
        pl.kernel,
        out_type=jax.ShapeDtypeStruct((N, D), jnp.float32),
        mesh=mesh,
        scratch_types=[
            pltpu.VMEM((CHUNK,), jnp.int32),
            pltpu.VMEM((CHUNK, D), jnp.float32),
            pltpu.VMEM((CHUNK, D), jnp.float32),
            pltpu.SemaphoreType.DMA,
            pltpu.SemaphoreType.DMA,
        ],
    )
    def sc_kernel(state_hbm, ts_hbm, table_hbm, out_hbm, idx_v, rows_v, st_v,
                  sem_g, sem_s):
        wid = lax.axis_index("s") * NC + lax.axis_index("c")
        base_w = wid * rows_per_w

        def chunk_body(ci, carry):
            base = base_w + ci * CHUNK
            pltpu.sync_copy(ts_hbm.at[pl.ds(base, CHUNK)], idx_v)
            g = pltpu.async_copy(table_hbm.at[idx_v], rows_v, sem_g)
            s = pltpu.async_copy(state_hbm.at[pl.ds(base, CHUNK), :], st_v,
                                 sem_s)
            g.wait()
            s.wait()

            def row_body(r, carry2):
                def vec_body(j, carry3):
                    sl = pl.ds(j * LANES, LANES)
                    rows_v[r, sl] = rows_v[r, sl] + st_v[r, sl]
                    return carry3

                return lax.fori_loop(0, D // LANES, vec_body, carry2)

            lax.fori_loop(0, CHUNK, row_body, 0)
            pltpu.sync_copy(rows_v, out_hbm.at[pl.ds(base, CHUNK), :])
            return carry

        lax.fori_loop(0, n_chunks, chunk_body, 0)

    out = sc_kernel(state_f, ts_f, embed_table)
    return out.reshape(B, L, D)


# speedup vs baseline: 1.5980x; 1.5980x over previous
import functools

import jax
import jax.numpy as jnp
from jax import lax
from jax.experimental import pallas as pl
from jax.experimental.pallas import tpu as pltpu
from jax.experimental.pallas import tpu_sc as plsc

NC, NS, LANES = 2, 16, 16
NW = NC * NS
CHUNK = 32


def kernel(state, timestep, embed_table):
    B, L, D = state.shape
    N = B * L
    state_f = state.reshape(N, D)
    ts_f = timestep.reshape(N)
    rows_per_w = N // NW
    n_chunks = rows_per_w // CHUNK

    mesh = plsc.VectorSubcoreMesh(core_axis_name="c", subcore_axis_name="s")

    @functools.partial(
        pl.kernel,
        out_type=jax.ShapeDtypeStruct((N, D), jnp.float32),
        mesh=mesh,
        scratch_types=[
            pltpu.VMEM((CHUNK,), jnp.int32),
            pltpu.VMEM((CHUNK, D), jnp.float32),
            pltpu.VMEM((CHUNK, D), jnp.float32),
            pltpu.SemaphoreType.DMA,
            pltpu.SemaphoreType.DMA,
        ],
    )
    def sc_kernel(state_hbm, ts_hbm, table_hbm, out_hbm, idx_v, rows_v, st_v,
                  sem_g, sem_s):
        wid = lax.axis_index("s") * NC + lax.axis_index("c")
        base_w = wid * rows_per_w

        def chunk_body(ci, carry):
            base = base_w + ci * CHUNK
            pltpu.sync_copy(ts_hbm.at[pl.ds(base, CHUNK)], idx_v)
            g = pltpu.async_copy(table_hbm.at[idx_v], rows_v, sem_g)
            s = pltpu.async_copy(state_hbm.at[pl.ds(base, CHUNK), :], st_v,
                                 sem_s)
            g.wait()
            s.wait()

            def row_body(r, carry2):
                for j in range(D // LANES):
                    sl = pl.ds(j * LANES, LANES)
                    rows_v[r, sl] = rows_v[r, sl] + st_v[r, sl]
                return carry2

            lax.fori_loop(0, CHUNK, row_body, 0)
            pltpu.sync_copy(rows_v, out_hbm.at[pl.ds(base, CHUNK), :])
            return carry

        lax.fori_loop(0, n_chunks, chunk_body, 0)

    out = sc_kernel(state_f, ts_f, embed_table)
    return out.reshape(B, L, D)


# speedup vs baseline: 2.1822x; 1.3656x over previous
import functools

import jax
import jax.numpy as jnp
from jax import lax
from jax.experimental import pallas as pl
from jax.experimental.pallas import tpu as pltpu
from jax.experimental.pallas import tpu_sc as plsc

NC, NS, LANES = 2, 16, 16
NW = NC * NS
CHUNK = 16
NBUF = 2


def kernel(state, timestep, embed_table):
    B, L, D = state.shape
    N = B * L
    state_f = state.reshape(N, D)
    ts_f = timestep.reshape(N)
    rows_per_w = N // NW
    n_chunks = rows_per_w // CHUNK
    n_groups = n_chunks // NBUF

    mesh = plsc.VectorSubcoreMesh(core_axis_name="c", subcore_axis_name="s")

    @functools.partial(
        pl.kernel,
        out_type=jax.ShapeDtypeStruct((N, D), jnp.float32),
        mesh=mesh,
        scratch_types=[
            pltpu.VMEM((rows_per_w,), jnp.int32),
            pltpu.VMEM((NBUF, CHUNK, D), jnp.float32),
            pltpu.VMEM((NBUF, CHUNK, D), jnp.float32),
            pltpu.VMEM((NBUF, CHUNK, D), jnp.float32),
            pltpu.SemaphoreType.DMA,
            pltpu.SemaphoreType.DMA,
            pltpu.SemaphoreType.DMA,
            pltpu.SemaphoreType.DMA,
            pltpu.SemaphoreType.DMA,
            pltpu.SemaphoreType.DMA,
        ],
    )
    def sc_kernel(state_hbm, ts_hbm, table_hbm, out_hbm, idx_v, rows_v, st_v,
                  ob_v, sg0, sg1, ss0, ss1, so0, so1):
        sem_g = (sg0, sg1)
        sem_s = (ss0, ss1)
        sem_o = (so0, so1)
        wid = lax.axis_index("s") * NC + lax.axis_index("c")
        base_w = wid * rows_per_w

        pltpu.sync_copy(ts_hbm.at[pl.ds(base_w, rows_per_w)], idx_v)

        def issue_loads(ci, b):
            base = base_w + ci * CHUNK
            pltpu.async_copy(table_hbm.at[idx_v.at[pl.ds(ci * CHUNK, CHUNK)]],
                             rows_v.at[b], sem_g[b])
            pltpu.async_copy(state_hbm.at[pl.ds(base, CHUNK), :],
                             st_v.at[b], sem_s[b])

        def wait_loads(ci, b):
            pltpu.make_async_copy(
                table_hbm.at[idx_v.at[pl.ds(ci * CHUNK, CHUNK)]],
                rows_v.at[b], sem_g[b]).wait()
            pltpu.make_async_copy(
                state_hbm.at[pl.ds(base_w + ci * CHUNK, CHUNK), :],
                st_v.at[b], sem_s[b]).wait()

        def do_add(b):
            def row_body(r, carry):
                for j in range(D // LANES):
                    sl = pl.ds(j * LANES, LANES)
                    ob_v[b, r, sl] = rows_v[b, r, sl] + st_v[b, r, sl]
                return carry

            lax.fori_loop(0, CHUNK, row_body, 0)

        def issue_out(ci, b):
            pltpu.async_copy(ob_v.at[b],
                             out_hbm.at[pl.ds(base_w + ci * CHUNK, CHUNK), :],
                             sem_o[b])

        def wait_out(ci, b):
            pltpu.make_async_copy(
                ob_v.at[b],
                out_hbm.at[pl.ds(base_w + ci * CHUNK, CHUNK), :],
                sem_o[b]).wait()

        for b in range(NBUF):
            issue_loads(b, b)

        def group_body(g, carry):
            for b in range(NBUF):
                ci = g * NBUF + b
                wait_loads(ci, b)

                @pl.when(g > 0)
                def _():
                    wait_out(ci - NBUF, b)

                do_add(b)
                issue_out(ci, b)

                @pl.when(g < n_groups - 1)
                def _():
                    issue_loads(ci + NBUF, b)

            return carry

        lax.fori_loop(0, n_groups, group_body, 0)

        for b in range(NBUF):
            wait_out(n_chunks - NBUF + b, b)

    out = sc_kernel(state_f, ts_f, embed_table)
    return out.reshape(B, L, D)


# speedup vs baseline: 2.7156x; 1.2444x over previous
import functools

import jax
import jax.numpy as jnp
from jax import lax
from jax.experimental import pallas as pl
from jax.experimental.pallas import tpu as pltpu
from jax.experimental.pallas import tpu_sc as plsc

NC, NS, LANES = 2, 16, 16
NW = NC * NS
CHUNK = 8
NBUF = 4


def kernel(state, timestep, embed_table):
    B, L, D = state.shape
    N = B * L
    state_f = state.reshape(N, D)
    ts_f = timestep.reshape(N)
    rows_per_w = N // NW
    n_chunks = rows_per_w // CHUNK
    n_groups = n_chunks // NBUF

    mesh = plsc.VectorSubcoreMesh(core_axis_name="c", subcore_axis_name="s")

    @functools.partial(
        pl.kernel,
        out_type=jax.ShapeDtypeStruct((N, D), jnp.float32),
        mesh=mesh,
        scratch_types=[
            pltpu.VMEM((rows_per_w,), jnp.int32),
            pltpu.VMEM((NBUF, CHUNK, D), jnp.float32),
            pltpu.VMEM((NBUF, CHUNK, D), jnp.float32),
            pltpu.VMEM((NBUF, CHUNK, D), jnp.float32),
        ] + [pltpu.SemaphoreType.DMA] * (3 * NBUF),
    )
    def sc_kernel(state_hbm, ts_hbm, table_hbm, out_hbm, idx_v, rows_v, st_v,
                  ob_v, *sems):
        sem_g = sems[0:NBUF]
        sem_s = sems[NBUF:2 * NBUF]
        sem_o = sems[2 * NBUF:3 * NBUF]
        wid = lax.axis_index("s") * NC + lax.axis_index("c")
        base_w = wid * rows_per_w

        pltpu.sync_copy(ts_hbm.at[pl.ds(base_w, rows_per_w)], idx_v)

        def issue_loads(ci, b):
            base = base_w + ci * CHUNK
            pltpu.async_copy(table_hbm.at[idx_v.at[pl.ds(ci * CHUNK, CHUNK)]],
                             rows_v.at[b], sem_g[b])
            pltpu.async_copy(state_hbm.at[pl.ds(base, CHUNK), :],
                             st_v.at[b], sem_s[b])

        def wait_loads(ci, b):
            pltpu.make_async_copy(
                table_hbm.at[idx_v.at[pl.ds(ci * CHUNK, CHUNK)]],
                rows_v.at[b], sem_g[b]).wait()
            pltpu.make_async_copy(
                state_hbm.at[pl.ds(base_w + ci * CHUNK, CHUNK), :],
                st_v.at[b], sem_s[b]).wait()

        def do_add(b):
            def row_body(r, carry):
                for j in range(D // LANES):
                    sl = pl.ds(j * LANES, LANES)
                    ob_v[b, r, sl] = rows_v[b, r, sl] + st_v[b, r, sl]
                return carry

            lax.fori_loop(0, CHUNK, row_body, 0)

        def issue_out(ci, b):
            pltpu.async_copy(ob_v.at[b],
                             out_hbm.at[pl.ds(base_w + ci * CHUNK, CHUNK), :],
                             sem_o[b])

        def wait_out(ci, b):
            pltpu.make_async_copy(
                ob_v.at[b],
                out_hbm.at[pl.ds(base_w + ci * CHUNK, CHUNK), :],
                sem_o[b]).wait()

        for b in range(NBUF):
            issue_loads(b, b)

        def group_body(g, carry):
            for b in range(NBUF):
                ci = g * NBUF + b
                wait_loads(ci, b)

                @pl.when(g > 0)
                def _():
                    wait_out(ci - NBUF, b)

                do_add(b)
                issue_out(ci, b)

                @pl.when(g < n_groups - 1)
                def _():
                    issue_loads(ci + NBUF, b)

            return carry

        lax.fori_loop(0, n_groups, group_body, 0)

        for b in range(NBUF):
            wait_out(n_chunks - NBUF + b, b)

    out = sc_kernel(state_f, ts_f, embed_table)
    return out.reshape(B, L, D)
